# Initial kernel scaffold; baseline (speedup 1.0000x reference)
import jax
import jax.numpy as jnp
from jax.experimental import pallas as pl


def kernel(time, pe):
    raise NotImplementedError("write your pallas kernel here")


# speedup vs baseline: 3.5183x; 3.5183x over previous
import functools

import jax
import jax.numpy as jnp
from jax import lax
from jax.experimental import pallas as pl
from jax.experimental.pallas import tpu as pltpu
from jax.experimental.pallas import tpu_sc as plsc

EMBED = 128
N_IDX = 4 * 8192
NW = 32
B_PER_W = N_IDX // NW
CHUNK = 128
N_CHUNKS = B_PER_W // CHUNK


@functools.partial(
    pl.kernel,
    out_type=jax.ShapeDtypeStruct((N_IDX, EMBED), jnp.float32),
    mesh=plsc.VectorSubcoreMesh(core_axis_name="c", subcore_axis_name="s"),
    scratch_types=[
        pltpu.VMEM((N_CHUNKS, CHUNK), jnp.int32),
        pltpu.VMEM((CHUNK, EMBED), jnp.float32),
        pltpu.VMEM((CHUNK, EMBED), jnp.float32),
        pltpu.SemaphoreType.DMA,
        pltpu.SemaphoreType.DMA,
    ],
)
def _gather_kernel(table_hbm, idx_hbm, out_hbm, idx_v, rows_a, rows_b, sem_g, sem_s):
    info = plsc.get_sparse_core_info()
    wid = lax.axis_index("s") * info.num_cores + lax.axis_index("c")

    pltpu.sync_copy(idx_hbm.at[wid], idx_v)

    base = wid * B_PER_W
    bufs = (rows_a, rows_b)

    def gather(j, buf):
        return pltpu.async_copy(table_hbm.at[idx_v.at[j]], buf, sem_g)

    def store(j, buf):
        return pltpu.async_copy(
            buf, out_hbm.at[pl.ds(base + j * CHUNK, CHUNK)], sem_s
        )

    g = gather(0, bufs[0])
    prev_s = None
    for j in range(N_CHUNKS):
        g.wait()
        if prev_s is not None:
            prev_s.wait()
        if j + 1 < N_CHUNKS:
            g = gather(j + 1, bufs[(j + 1) % 2])
        prev_s = store(j, bufs[j % 2])
    prev_s.wait()


def kernel(time, pe):
    idx = time.astype(jnp.int32).reshape(NW, N_CHUNKS, CHUNK)
    flat = _gather_kernel(pe, idx)
    return flat.reshape(time.shape + (EMBED,))


# speedup vs baseline: 3.6789x; 1.0457x over previous
import functools

import jax
import jax.numpy as jnp
from jax import lax
from jax.experimental import pallas as pl
from jax.experimental.pallas import tpu as pltpu
from jax.experimental.pallas import tpu_sc as plsc

EMBED = 128
N_IDX = 4 * 8192
NW = 32
B_PER_W = N_IDX // NW
CHUNK = 128
N_CHUNKS = B_PER_W // CHUNK


DEPTH = 4


@functools.partial(
    pl.kernel,
    out_type=jax.ShapeDtypeStruct((N_IDX, EMBED), jnp.float32),
    mesh=plsc.VectorSubcoreMesh(core_axis_name="c", subcore_axis_name="s"),
    scratch_types=[
        pltpu.VMEM((N_CHUNKS, CHUNK), jnp.int32),
        pltpu.VMEM((DEPTH, CHUNK, EMBED), jnp.float32),
        pltpu.SemaphoreType.DMA,
        pltpu.SemaphoreType.DMA,
    ],
)
def _gather_kernel(table_hbm, idx_hbm, out_hbm, idx_v, rows_v, sem_g, sem_s):
    info = plsc.get_sparse_core_info()
    wid = lax.axis_index("s") * info.num_cores + lax.axis_index("c")

    pltpu.sync_copy(idx_hbm.at[wid], idx_v)

    base = wid * B_PER_W

    def gather(j):
        return pltpu.async_copy(
            table_hbm.at[idx_v.at[j]], rows_v.at[j % DEPTH], sem_g
        )

    def store(j):
        return pltpu.async_copy(
            rows_v.at[j % DEPTH], out_hbm.at[pl.ds(base + j * CHUNK, CHUNK)], sem_s
        )

    g = [None] * N_CHUNKS
    s = [None] * N_CHUNKS
    for j in range(min(DEPTH, N_CHUNKS)):
        g[j] = gather(j)
    for j in range(N_CHUNKS):
        g[j].wait()
        s[j] = store(j)
        k = j + 1 - DEPTH
        if k >= 0 and k + DEPTH < N_CHUNKS:
            s[k].wait()
            g[k + DEPTH] = gather(k + DEPTH)
    for j in range(max(0, N_CHUNKS - DEPTH), N_CHUNKS):
        s[j].wait()


def kernel(time, pe):
    idx = time.astype(jnp.int32).reshape(NW, N_CHUNKS, CHUNK)
    flat = _gather_kernel(pe, idx)
    return flat.reshape(time.shape + (EMBED,))
